# Initial kernel scaffold; baseline (speedup 1.0000x reference)
import functools

import jax
import jax.numpy as jnp
from jax import lax
from jax.experimental import pallas as pl
from jax.experimental.pallas import tpu as pltpu
from jax.experimental.pallas import tpu_sc as plsc

_K = 64
_ROWS = 8


def _topk_block(x_ref, vals_ref, idx_ref):
    x = x_ref[...]
    n = x.shape[1]
    cols = lax.broadcasted_iota(jnp.int32, x.shape, 1)
    vals = []
    idxs = []
    for _ in range(_K):
        m = jnp.max(x, axis=1, keepdims=True)
        sel = jnp.min(jnp.where(x == m, cols, n), axis=1, keepdims=True)
        vals.append(m)
        idxs.append(sel)
        x = jnp.where(cols == sel, -jnp.inf, x)
    vals_ref[...] = jnp.concatenate(vals, axis=1)
    idx_ref[...] = jnp.concatenate(idxs, axis=1)


def _topk(em2d):
    r, v = em2d.shape
    return pl.pallas_call(
        _topk_block,
        grid=(r // _ROWS,),
        in_specs=[pl.BlockSpec((_ROWS, v), lambda i: (i, 0))],
        out_specs=[
            pl.BlockSpec((_ROWS, _K), lambda i: (i, 0)),
            pl.BlockSpec((_ROWS, _K), lambda i: (i, 0)),
        ],
        out_shape=[
            jax.ShapeDtypeStruct((r, _K), jnp.float32),
            jax.ShapeDtypeStruct((r, _K), jnp.int32),
        ],
    )(em2d)


def _sc_gather(table, idx_flat):
    info = plsc.get_sparse_core_info()
    nc, ns = info.num_cores, info.num_subcores
    nw = nc * ns
    n_idx = idx_flat.shape[0]
    d = table.shape[1]
    per_w = n_idx // nw
    chunks = per_w // 128
    idx3 = idx_flat.reshape(nw, chunks, 128)
    mesh = plsc.VectorSubcoreMesh(core_axis_name="c", subcore_axis_name="s")

    @functools.partial(
        pl.kernel,
        mesh=mesh,
        out_type=jax.ShapeDtypeStruct((n_idx, d), jnp.float32),
        scratch_types=[
            pltpu.VMEM((chunks, 128), jnp.int32),
            pltpu.VMEM((per_w, d), jnp.float32),
            pltpu.SemaphoreType.DMA,
        ],
    )
    def gather_kernel(table_hbm, idx_hbm, out_hbm, idx_v, rows_v, sem):
        wid = lax.axis_index("s") * nc + lax.axis_index("c")
        pltpu.sync_copy(idx_hbm.at[wid], idx_v)
        copies = [
            pltpu.async_copy(
                table_hbm.at[idx_v.at[j]],
                rows_v.at[pl.ds(j * 128, 128)],
                sem,
            )
            for j in range(chunks)
        ]
        for cp in copies:
            cp.wait()
        pltpu.sync_copy(rows_v, out_hbm.at[pl.ds(wid * per_w, per_w)])

    return gather_kernel(table, idx3)


def _crf_block(emis_ref, st_ref, len_ref, out_ref):
    emis = emis_ref[0]
    t_len = emis.shape[0]
    seq_len = len_ref[0, 0, 0]
    ii = lax.broadcasted_iota(jnp.int32, (_K, _K), 0)
    jj = lax.broadcasted_iota(jnp.int32, (_K, _K), 1)
    eye = (ii == jj).astype(jnp.float32)

    def lse_row(a):
        m = jnp.max(a, axis=1, keepdims=True)
        return m + jnp.log(jnp.sum(jnp.exp(a - m), axis=1, keepdims=True))

    def to_col(row):
        return lax.dot_general(eye, row, (((1,), (1,)), ((), ())),
                               preferred_element_type=jnp.float32)

    alpha_row = emis[0:1, :]
    res = lse_row(alpha_row)
    alpha_col = to_col(alpha_row)
    for t in range(1, t_len):
        trans = lax.dot_general(st_ref[0, t - 1], st_ref[0, t],
                                (((1,), (1,)), ((), ())),
                                preferred_element_type=jnp.float32)
        mat = trans + alpha_col
        m = jnp.max(mat, axis=0, keepdims=True)
        new_row = (m + jnp.log(jnp.sum(jnp.exp(mat - m), axis=0, keepdims=True))
                   + emis[t:t + 1, :])
        res = jnp.where(seq_len - 1 == t, lse_row(new_row), res)
        alpha_col = to_col(new_row)
    out_ref[0, 0, 0] = res[0, 0]


def _crf(emis3, st4, lens3):
    b, t, _ = emis3.shape
    return pl.pallas_call(
        _crf_block,
        grid=(b,),
        in_specs=[
            pl.BlockSpec((1, t, _K), lambda i: (i, 0, 0)),
            pl.BlockSpec((1, t, _K, _K), lambda i: (i, 0, 0, 0)),
            pl.BlockSpec((1, 1, 1), lambda i: (i, 0, 0)),
        ],
        out_specs=pl.BlockSpec((1, 1, 1), lambda i: (i, 0, 0)),
        out_shape=jax.ShapeDtypeStruct((b, 1, 1), jnp.float32),
    )(emis3, st4, lens3)


def kernel(state_matrix, emission_potentials, seq_lens, sum_size):
    b, t, v = emission_potentials.shape
    em2d = emission_potentials.reshape(b * t, v)
    vals, idx = _topk(em2d)
    sampled = _sc_gather(state_matrix, idx.reshape(-1))
    emis3 = vals.reshape(b, t, _K)
    st4 = sampled.reshape(b, t, _K, state_matrix.shape[1])
    lens3 = seq_lens.reshape(b, 1, 1)
    out3 = _crf(emis3, st4, lens3)
    return out3.reshape(b)


# speedup vs baseline: 3.5646x; 3.5646x over previous
import functools

import jax
import jax.numpy as jnp
from jax import lax
from jax.experimental import pallas as pl
from jax.experimental.pallas import tpu as pltpu
from jax.experimental.pallas import tpu_sc as plsc

_K = 64
_ROWS = 8


def _topk_block(x_ref, vals_ref, idx_ref):
    x = x_ref[...]
    n = x.shape[1]
    cols = lax.broadcasted_iota(jnp.int32, x.shape, 1)
    vals = []
    idxs = []
    for _ in range(_K):
        m = jnp.max(x, axis=1, keepdims=True)
        sel = jnp.min(jnp.where(x == m, cols, n), axis=1, keepdims=True)
        vals.append(m)
        idxs.append(sel)
        x = jnp.where(cols == sel, -jnp.inf, x)
    vals_ref[...] = jnp.concatenate(vals, axis=1)
    idx_ref[...] = jnp.concatenate(idxs, axis=1)


def _topk(em2d):
    r, v = em2d.shape
    return pl.pallas_call(
        _topk_block,
        grid=(r // _ROWS,),
        in_specs=[pl.BlockSpec((_ROWS, v), lambda i: (i, 0))],
        out_specs=[
            pl.BlockSpec((_ROWS, _K), lambda i: (i, 0)),
            pl.BlockSpec((_ROWS, _K), lambda i: (i, 0)),
        ],
        out_shape=[
            jax.ShapeDtypeStruct((r, _K), jnp.float32),
            jax.ShapeDtypeStruct((r, _K), jnp.int32),
        ],
    )(em2d)


def _sc_gather(table, idx_flat):
    info = plsc.get_sparse_core_info()
    nc, ns = info.num_cores, info.num_subcores
    nw = nc * ns
    n_idx = idx_flat.shape[0]
    d = table.shape[1]
    per_w = n_idx // nw
    chunks = per_w // 128
    half = chunks // 2
    idx3 = idx_flat.reshape(nw, chunks, 128)
    mesh = plsc.VectorSubcoreMesh(core_axis_name="c", subcore_axis_name="s")

    @functools.partial(
        pl.kernel,
        mesh=mesh,
        out_type=jax.ShapeDtypeStruct((n_idx, d), jnp.float32),
        scratch_types=[
            pltpu.VMEM((chunks, 128), jnp.int32),
            pltpu.VMEM((half * 128, d), jnp.float32),
            pltpu.SemaphoreType.DMA,
        ],
    )
    def gather_kernel(table_hbm, idx_hbm, out_hbm, idx_v, rows_v, sem):
        wid = lax.axis_index("s") * nc + lax.axis_index("c")
        pltpu.sync_copy(idx_hbm.at[wid], idx_v)
        for h in range(2):
            copies = [
                pltpu.async_copy(
                    table_hbm.at[idx_v.at[h * half + j]],
                    rows_v.at[pl.ds(j * 128, 128)],
                    sem,
                )
                for j in range(half)
            ]
            for cp in copies:
                cp.wait()
            pltpu.sync_copy(
                rows_v,
                out_hbm.at[pl.ds(wid * per_w + h * half * 128, half * 128)],
            )

    return gather_kernel(table, idx3)


def _crf_block(emis_ref, st_ref, len_ref, out_ref):
    emis = emis_ref[0]
    t_len = emis.shape[0]
    seq_len = len_ref[0, 0, 0]
    ii = lax.broadcasted_iota(jnp.int32, (_K, _K), 0)
    jj = lax.broadcasted_iota(jnp.int32, (_K, _K), 1)
    eye = (ii == jj).astype(jnp.float32)

    def lse_row(a):
        m = jnp.max(a, axis=1, keepdims=True)
        return m + jnp.log(jnp.sum(jnp.exp(a - m), axis=1, keepdims=True))

    def to_col(row):
        return lax.dot_general(eye, row, (((1,), (1,)), ((), ())),
                               preferred_element_type=jnp.float32)

    alpha_row = emis[0:1, :]
    res = lse_row(alpha_row)
    alpha_col = to_col(alpha_row)
    for t in range(1, t_len):
        trans = lax.dot_general(st_ref[0, t - 1], st_ref[0, t],
                                (((1,), (1,)), ((), ())),
                                preferred_element_type=jnp.float32)
        mat = trans + alpha_col
        m = jnp.max(mat, axis=0, keepdims=True)
        new_row = (m + jnp.log(jnp.sum(jnp.exp(mat - m), axis=0, keepdims=True))
                   + emis[t:t + 1, :])
        res = jnp.where(seq_len - 1 == t, lse_row(new_row), res)
        alpha_col = to_col(new_row)
    out_ref[...] = res.reshape(1, 1, 1)


def _crf(emis3, st4, lens3):
    b, t, _ = emis3.shape
    return pl.pallas_call(
        _crf_block,
        grid=(b,),
        in_specs=[
            pl.BlockSpec((1, t, _K), lambda i: (i, 0, 0)),
            pl.BlockSpec((1, t, _K, _K), lambda i: (i, 0, 0, 0)),
            pl.BlockSpec((1, 1, 1), lambda i: (i, 0, 0)),
        ],
        out_specs=pl.BlockSpec((1, 1, 1), lambda i: (i, 0, 0)),
        out_shape=jax.ShapeDtypeStruct((b, 1, 1), jnp.float32),
    )(emis3, st4, lens3)


def kernel(state_matrix, emission_potentials, seq_lens, sum_size):
    b, t, v = emission_potentials.shape
    em2d = emission_potentials.reshape(b * t, v)
    vals, idx = _topk(em2d)
    d = state_matrix.shape[1]
    table_pad = jnp.pad(state_matrix, ((0, 0), (0, 128 - d)))
    sampled = _sc_gather(table_pad, idx.reshape(-1))[:, :d]
    emis3 = vals.reshape(b, t, _K)
    st4 = sampled.reshape(b, t, _K, d)
    lens3 = seq_lens.reshape(b, 1, 1)
    out3 = _crf(emis3, st4, lens3)
    return out3.reshape(b)


# speedup vs baseline: 4.8914x; 1.3722x over previous
import functools

import jax
import jax.numpy as jnp
from jax import lax
from jax.experimental import pallas as pl
from jax.experimental.pallas import tpu as pltpu
from jax.experimental.pallas import tpu_sc as plsc

_K = 64
_ROWS = 8


def _topk_block(x_ref, vals_ref, idx_ref):
    x = x_ref[...]
    n = x.shape[1]
    cols = lax.broadcasted_iota(jnp.int32, x.shape, 1)
    vals = []
    idxs = []
    for _ in range(_K):
        m = jnp.max(x, axis=1, keepdims=True)
        sel = jnp.min(jnp.where(x == m, cols, n), axis=1, keepdims=True)
        vals.append(m)
        idxs.append(sel)
        x = jnp.where(cols == sel, -jnp.inf, x)
    vals_ref[...] = jnp.concatenate(vals, axis=1)
    idx_ref[...] = jnp.concatenate(idxs, axis=1)


def _topk(em2d):
    r, v = em2d.shape
    return pl.pallas_call(
        _topk_block,
        grid=(r // _ROWS,),
        in_specs=[pl.BlockSpec((_ROWS, v), lambda i: (i, 0))],
        out_specs=[
            pl.BlockSpec((_ROWS, _K), lambda i: (i, 0)),
            pl.BlockSpec((_ROWS, _K), lambda i: (i, 0)),
        ],
        out_shape=[
            jax.ShapeDtypeStruct((r, _K), jnp.float32),
            jax.ShapeDtypeStruct((r, _K), jnp.int32),
        ],
    )(em2d)


_CAP = 2048

_GDN = lax.GatherDimensionNumbers(
    offset_dims=(), collapsed_slice_dims=(0,), start_index_map=(0,))


def _gather16(v, idx):
    return lax.gather(v, idx[:, None], _GDN, (1,),
                      mode=lax.GatherScatterMode.PROMISE_IN_BOUNDS)


def _sc_compact(em2d):
    rows, v = em2d.shape
    info = plsc.get_sparse_core_info()
    nc, ns = info.num_cores, info.num_subcores
    nw = nc * ns
    rows_per_w = rows // nw
    nvec = v // 16
    mesh = plsc.VectorSubcoreMesh(core_axis_name="c", subcore_axis_name="s")

    @functools.partial(
        pl.kernel,
        mesh=mesh,
        out_type=[
            jax.ShapeDtypeStruct((rows, _CAP), jnp.float32),
            jax.ShapeDtypeStruct((rows, _CAP), jnp.int32),
        ],
        scratch_types=[
            pltpu.VMEM((v,), jnp.float32),
            pltpu.VMEM((_CAP + 16,), jnp.float32),
            pltpu.VMEM((_CAP + 16,), jnp.int32),
        ],
        compiler_params=pltpu.CompilerParams(needs_layout_passes=False),
    )
    def compact_kernel(em_hbm, val_hbm, pos_hbm, row_buf, val_buf, pos_buf):
        wid = lax.axis_index("s") * nc + lax.axis_index("c")
        lane = lax.broadcasted_iota(jnp.int32, (16,), 0)
        neg_inf = jnp.full((16,), -jnp.inf, jnp.float32)

        def lane_max(vec):
            m = lax.squeeze(lax.slice(vec, [0], [1]), [0])
            for l in range(1, 16):
                m = jnp.maximum(m, lax.squeeze(lax.slice(vec, [l], [l + 1]), [0]))
            return m

        def row_body(r, carry):
            row = wid * rows_per_w + r
            pltpu.sync_copy(em_hbm.at[row], row_buf)

            def chunk_body(c, t0):
                def vmax_body(i, acc):
                    return jnp.maximum(acc, row_buf[pl.ds(c * 512 + i * 16, 16)])
                acc = lax.fori_loop(0, 32, vmax_body, neg_inf)
                return jnp.minimum(t0, lane_max(acc))

            t0 = lax.fori_loop(0, 64, chunk_body, jnp.float32(jnp.inf))

            def init_body(i, carry2):
                val_buf[pl.ds(i * 16, 16)] = neg_inf
                return carry2

            lax.fori_loop(0, (_CAP + 16) // 16, init_body, 0)

            def comp_body(i, cnt):
                x = row_buf[pl.ds(i * 16, 16)]
                msk = x >= t0
                pc = plsc.all_reduce_population_count(msk)
                pcs = lax.squeeze(lax.slice(pc, [0], [1]), [0])

                def do_store(cnt_in):
                    y = msk.astype(jnp.int32)
                    for s in (1, 2, 4, 8):
                        g = _gather16(y, jnp.maximum(lane - s, 0))
                        y = y + jnp.where(lane >= s, g, 0)
                    posv = jnp.where(msk, cnt_in + y - 1, _CAP + 15)
                    plsc.store_scatter(val_buf, [posv], x)
                    plsc.store_scatter(pos_buf, [posv], lane + i * 16)
                    return jnp.minimum(cnt_in + pcs, _CAP)

                return lax.cond(pcs > 0, do_store, lambda c: c, cnt)

            lax.fori_loop(0, nvec, comp_body, jnp.int32(0))

            pltpu.sync_copy(val_buf.at[pl.ds(0, _CAP)], val_hbm.at[row])
            pltpu.sync_copy(pos_buf.at[pl.ds(0, _CAP)], pos_hbm.at[row])
            return carry

        lax.fori_loop(0, rows_per_w, row_body, 0)

    return compact_kernel(em2d)


def _sc_gather(table, cand_pos, sel_pos):
    info = plsc.get_sparse_core_info()
    nc, ns = info.num_cores, info.num_subcores
    nw = nc * ns
    rows, cap = cand_pos.shape
    n_idx = sel_pos.size
    d = table.shape[1]
    per_w = n_idx // nw
    rows_per_w = rows // nw
    k = n_idx // rows
    chunks = per_w // 128
    half = chunks // 2
    sel_flat = sel_pos.reshape(-1)
    mesh = plsc.VectorSubcoreMesh(core_axis_name="c", subcore_axis_name="s")

    @functools.partial(
        pl.kernel,
        mesh=mesh,
        out_type=jax.ShapeDtypeStruct((n_idx, d), jnp.float32),
        scratch_types=[
            pltpu.VMEM((rows_per_w, cap), jnp.int32),
            pltpu.VMEM((per_w,), jnp.int32),
            pltpu.VMEM((chunks, 128), jnp.int32),
            pltpu.VMEM((half * 128, d), jnp.float32),
            pltpu.SemaphoreType.DMA,
        ],
        compiler_params=pltpu.CompilerParams(needs_layout_passes=False),
    )
    def gather_kernel(table_hbm, cpos_hbm, sel_hbm, out_hbm,
                      cpos_v, sel_v, idx_v, rows_v, sem):
        wid = lax.axis_index("s") * nc + lax.axis_index("c")
        lane = lax.broadcasted_iota(jnp.int32, (16,), 0)
        pltpu.sync_copy(cpos_hbm.at[pl.ds(wid * rows_per_w, rows_per_w)], cpos_v)
        pltpu.sync_copy(sel_hbm.at[pl.ds(wid * per_w, per_w)], sel_v)
        kshift = k.bit_length() - 1
        for j in range(per_w // 16):
            slot = lane + j * 16
            row_loc = lax.shift_right_logical(slot, kshift)
            pos_v = sel_v[pl.ds(j * 16, 16)]
            real = plsc.load_gather(cpos_v, [row_loc, pos_v])
            idx_v[j // 8, pl.ds((j % 8) * 16, 16)] = real
        for h in range(2):
            copies = [
                pltpu.async_copy(
                    table_hbm.at[idx_v.at[h * half + j]],
                    rows_v.at[pl.ds(j * 128, 128)],
                    sem,
                )
                for j in range(half)
            ]
            for cp in copies:
                cp.wait()
            pltpu.sync_copy(
                rows_v,
                out_hbm.at[pl.ds(wid * per_w + h * half * 128, half * 128)],
            )

    return gather_kernel(table, cand_pos, sel_flat)


def _crf_block(emis_ref, st_ref, len_ref, out_ref):
    emis = emis_ref[0]
    t_len = emis.shape[0]
    seq_len = len_ref[0, 0, 0]
    ii = lax.broadcasted_iota(jnp.int32, (_K, _K), 0)
    jj = lax.broadcasted_iota(jnp.int32, (_K, _K), 1)
    eye = (ii == jj).astype(jnp.float32)

    def lse_row(a):
        m = jnp.max(a, axis=1, keepdims=True)
        return m + jnp.log(jnp.sum(jnp.exp(a - m), axis=1, keepdims=True))

    def to_col(row):
        return lax.dot_general(eye, row, (((1,), (1,)), ((), ())),
                               preferred_element_type=jnp.float32)

    alpha_row = emis[0:1, :]
    res = lse_row(alpha_row)
    alpha_col = to_col(alpha_row)
    for t in range(1, t_len):
        trans = lax.dot_general(st_ref[0, t - 1], st_ref[0, t],
                                (((1,), (1,)), ((), ())),
                                preferred_element_type=jnp.float32)
        mat = trans + alpha_col
        m = jnp.max(mat, axis=0, keepdims=True)
        new_row = (m + jnp.log(jnp.sum(jnp.exp(mat - m), axis=0, keepdims=True))
                   + emis[t:t + 1, :])
        res = jnp.where(seq_len - 1 == t, lse_row(new_row), res)
        alpha_col = to_col(new_row)
    out_ref[...] = res.reshape(1, 1, 1)


def _crf(emis3, st4, lens3):
    b, t, _ = emis3.shape
    return pl.pallas_call(
        _crf_block,
        grid=(b,),
        in_specs=[
            pl.BlockSpec((1, t, _K), lambda i: (i, 0, 0)),
            pl.BlockSpec((1, t, _K, _K), lambda i: (i, 0, 0, 0)),
            pl.BlockSpec((1, 1, 1), lambda i: (i, 0, 0)),
        ],
        out_specs=pl.BlockSpec((1, 1, 1), lambda i: (i, 0, 0)),
        out_shape=jax.ShapeDtypeStruct((b, 1, 1), jnp.float32),
    )(emis3, st4, lens3)


def kernel(state_matrix, emission_potentials, seq_lens, sum_size):
    b, t, v = emission_potentials.shape
    em2d = emission_potentials.reshape(b * t, v)
    cand_val, cand_pos = _sc_compact(em2d)
    vals, sel_pos = _topk(cand_val)
    d = state_matrix.shape[1]
    table_pad = jnp.pad(state_matrix, ((0, 0), (0, 128 - d)))
    sampled = _sc_gather(table_pad, cand_pos, sel_pos)[:, :d]
    emis3 = vals.reshape(b, t, _K)
    st4 = sampled.reshape(b, t, _K, d)
    lens3 = seq_lens.reshape(b, 1, 1)
    out3 = _crf(emis3, st4, lens3)
    return out3.reshape(b)


# speedup vs baseline: 7.0363x; 1.4385x over previous
import functools

import jax
import jax.numpy as jnp
from jax import lax
from jax.experimental import pallas as pl
from jax.experimental.pallas import tpu as pltpu
from jax.experimental.pallas import tpu_sc as plsc

_K = 64
_ROWS = 8


def _topk_block(x_ref, vals_ref, idx_ref):
    x = x_ref[...]
    n = x.shape[1]
    cols = lax.broadcasted_iota(jnp.int32, x.shape, 1)
    vals = []
    idxs = []
    for _ in range(_K):
        m = jnp.max(x, axis=1, keepdims=True)
        sel = jnp.min(jnp.where(x == m, cols, n), axis=1, keepdims=True)
        vals.append(m)
        idxs.append(sel)
        x = jnp.where(cols == sel, -jnp.inf, x)
    vals_ref[...] = jnp.concatenate(vals, axis=1)
    idx_ref[...] = jnp.concatenate(idxs, axis=1)


def _topk(em2d):
    r, v = em2d.shape
    return pl.pallas_call(
        _topk_block,
        grid=(r // _ROWS,),
        in_specs=[pl.BlockSpec((_ROWS, v), lambda i: (i, 0))],
        out_specs=[
            pl.BlockSpec((_ROWS, _K), lambda i: (i, 0)),
            pl.BlockSpec((_ROWS, _K), lambda i: (i, 0)),
        ],
        out_shape=[
            jax.ShapeDtypeStruct((r, _K), jnp.float32),
            jax.ShapeDtypeStruct((r, _K), jnp.int32),
        ],
    )(em2d)


_CAP = 2048

_GDN = lax.GatherDimensionNumbers(
    offset_dims=(), collapsed_slice_dims=(0,), start_index_map=(0,))


def _gather16(v, idx):
    return lax.gather(v, idx[:, None], _GDN, (1,),
                      mode=lax.GatherScatterMode.PROMISE_IN_BOUNDS)


def _sc_compact(em2d):
    rows, v = em2d.shape
    info = plsc.get_sparse_core_info()
    nc, ns = info.num_cores, info.num_subcores
    nw = nc * ns
    rows_per_w = rows // nw
    nvec = v // 16
    mesh = plsc.VectorSubcoreMesh(core_axis_name="c", subcore_axis_name="s")

    @functools.partial(
        pl.kernel,
        mesh=mesh,
        out_type=[
            jax.ShapeDtypeStruct((rows, _CAP), jnp.float32),
            jax.ShapeDtypeStruct((rows, _CAP), jnp.int32),
        ],
        scratch_types=[
            pltpu.VMEM((v,), jnp.float32),
            pltpu.VMEM((_CAP + 16,), jnp.float32),
            pltpu.VMEM((_CAP + 16,), jnp.int32),
        ],
        compiler_params=pltpu.CompilerParams(needs_layout_passes=False),
    )
    def compact_kernel(em_hbm, val_hbm, pos_hbm, row_buf, val_buf, pos_buf):
        wid = lax.axis_index("s") * nc + lax.axis_index("c")
        lane = lax.broadcasted_iota(jnp.int32, (16,), 0)
        neg_inf = jnp.full((16,), -jnp.inf, jnp.float32)

        def bmax(vec):
            for s in (1, 2, 4, 8):
                vec = jnp.maximum(vec, _gather16(vec, jnp.bitwise_xor(lane, s)))
            return vec

        def row_body(r, carry):
            row = wid * rows_per_w + r
            pltpu.sync_copy(em_hbm.at[row], row_buf)

            def chunk_body(c, t0):
                acc = neg_inf
                for i in range(32):
                    acc = jnp.maximum(acc, row_buf[pl.ds(c * 512 + i * 16, 16)])
                return jnp.minimum(t0, bmax(acc))

            t0 = lax.fori_loop(0, 64, chunk_body,
                               jnp.full((16,), jnp.inf, jnp.float32))

            def init_body(i, carry2):
                val_buf[pl.ds(i * 16, 16)] = neg_inf
                return carry2

            lax.fori_loop(0, (_CAP + 16) // 16, init_body, 0)

            def comp_body(g, cnt_vec):
                for u in range(4):
                    i = g * 4 + u
                    x = row_buf[pl.ds(i * 16, 16)]
                    msk = x >= t0
                    posv = jnp.where(msk, lax.shift_left(cnt_vec, 4) + lane,
                                     _CAP + 15)
                    plsc.store_scatter(val_buf, [posv], x)
                    plsc.store_scatter(pos_buf, [posv], lane + i * 16)
                    cnt_vec = jnp.minimum(cnt_vec + msk.astype(jnp.int32),
                                          _CAP // 16 - 1)
                return cnt_vec

            lax.fori_loop(0, nvec // 4, comp_body, jnp.zeros((16,), jnp.int32))

            pltpu.sync_copy(val_buf.at[pl.ds(0, _CAP)], val_hbm.at[row])
            pltpu.sync_copy(pos_buf.at[pl.ds(0, _CAP)], pos_hbm.at[row])
            return carry

        lax.fori_loop(0, rows_per_w, row_body, 0)

    return compact_kernel(em2d)


def _sc_gather(table, cand_pos, sel_pos):
    info = plsc.get_sparse_core_info()
    nc, ns = info.num_cores, info.num_subcores
    nw = nc * ns
    rows, cap = cand_pos.shape
    n_idx = sel_pos.size
    d = table.shape[1]
    per_w = n_idx // nw
    rows_per_w = rows // nw
    k = n_idx // rows
    chunks = per_w // 128
    half = chunks // 2
    sel_flat = sel_pos.reshape(-1)
    mesh = plsc.VectorSubcoreMesh(core_axis_name="c", subcore_axis_name="s")

    @functools.partial(
        pl.kernel,
        mesh=mesh,
        out_type=jax.ShapeDtypeStruct((n_idx, d), jnp.float32),
        scratch_types=[
            pltpu.VMEM((rows_per_w, cap), jnp.int32),
            pltpu.VMEM((per_w,), jnp.int32),
            pltpu.VMEM((chunks, 128), jnp.int32),
            pltpu.VMEM((half * 128, d), jnp.float32),
            pltpu.SemaphoreType.DMA,
        ],
        compiler_params=pltpu.CompilerParams(needs_layout_passes=False),
    )
    def gather_kernel(table_hbm, cpos_hbm, sel_hbm, out_hbm,
                      cpos_v, sel_v, idx_v, rows_v, sem):
        wid = lax.axis_index("s") * nc + lax.axis_index("c")
        lane = lax.broadcasted_iota(jnp.int32, (16,), 0)
        pltpu.sync_copy(cpos_hbm.at[pl.ds(wid * rows_per_w, rows_per_w)], cpos_v)
        pltpu.sync_copy(sel_hbm.at[pl.ds(wid * per_w, per_w)], sel_v)
        kshift = k.bit_length() - 1
        for j in range(per_w // 16):
            slot = lane + j * 16
            row_loc = lax.shift_right_logical(slot, kshift)
            pos_v = sel_v[pl.ds(j * 16, 16)]
            real = plsc.load_gather(cpos_v, [row_loc, pos_v])
            idx_v[j // 8, pl.ds((j % 8) * 16, 16)] = real
        for h in range(2):
            copies = [
                pltpu.async_copy(
                    table_hbm.at[idx_v.at[h * half + j]],
                    rows_v.at[pl.ds(j * 128, 128)],
                    sem,
                )
                for j in range(half)
            ]
            for cp in copies:
                cp.wait()
            pltpu.sync_copy(
                rows_v,
                out_hbm.at[pl.ds(wid * per_w + h * half * 128, half * 128)],
            )

    return gather_kernel(table, cand_pos, sel_flat)


def _crf_block(emis_ref, st_ref, len_ref, out_ref):
    emis = emis_ref[0]
    t_len = emis.shape[0]
    seq_len = len_ref[0, 0, 0]
    ii = lax.broadcasted_iota(jnp.int32, (_K, _K), 0)
    jj = lax.broadcasted_iota(jnp.int32, (_K, _K), 1)
    eye = (ii == jj).astype(jnp.float32)

    def lse_row(a):
        m = jnp.max(a, axis=1, keepdims=True)
        return m + jnp.log(jnp.sum(jnp.exp(a - m), axis=1, keepdims=True))

    def to_col(row):
        return lax.dot_general(eye, row, (((1,), (1,)), ((), ())),
                               preferred_element_type=jnp.float32)

    alpha_row = emis[0:1, :]
    res = lse_row(alpha_row)
    alpha_col = to_col(alpha_row)
    for t in range(1, t_len):
        trans = lax.dot_general(st_ref[0, t - 1], st_ref[0, t],
                                (((1,), (1,)), ((), ())),
                                preferred_element_type=jnp.float32)
        mat = trans + alpha_col
        m = jnp.max(mat, axis=0, keepdims=True)
        new_row = (m + jnp.log(jnp.sum(jnp.exp(mat - m), axis=0, keepdims=True))
                   + emis[t:t + 1, :])
        res = jnp.where(seq_len - 1 == t, lse_row(new_row), res)
        alpha_col = to_col(new_row)
    out_ref[...] = res.reshape(1, 1, 1)


def _crf(emis3, st4, lens3):
    b, t, _ = emis3.shape
    return pl.pallas_call(
        _crf_block,
        grid=(b,),
        in_specs=[
            pl.BlockSpec((1, t, _K), lambda i: (i, 0, 0)),
            pl.BlockSpec((1, t, _K, _K), lambda i: (i, 0, 0, 0)),
            pl.BlockSpec((1, 1, 1), lambda i: (i, 0, 0)),
        ],
        out_specs=pl.BlockSpec((1, 1, 1), lambda i: (i, 0, 0)),
        out_shape=jax.ShapeDtypeStruct((b, 1, 1), jnp.float32),
    )(emis3, st4, lens3)


def kernel(state_matrix, emission_potentials, seq_lens, sum_size):
    b, t, v = emission_potentials.shape
    em2d = emission_potentials.reshape(b * t, v)
    cand_val, cand_pos = _sc_compact(em2d)
    vals, sel_pos = _topk(cand_val)
    d = state_matrix.shape[1]
    table_pad = jnp.pad(state_matrix, ((0, 0), (0, 128 - d)))
    sampled = _sc_gather(table_pad, cand_pos, sel_pos)[:, :d]
    emis3 = vals.reshape(b, t, _K)
    st4 = sampled.reshape(b, t, _K, d)
    lens3 = seq_lens.reshape(b, 1, 1)
    out3 = _crf(emis3, st4, lens3)
    return out3.reshape(b)


# speedup vs baseline: 7.2052x; 1.0240x over previous
import functools

import jax
import jax.numpy as jnp
from jax import lax
from jax.experimental import pallas as pl
from jax.experimental.pallas import tpu as pltpu
from jax.experimental.pallas import tpu_sc as plsc

_K = 64
_ROWS = 8


def _topk_block(x_ref, vals_ref, idx_ref):
    x = x_ref[...]
    n = x.shape[1]
    cols = lax.broadcasted_iota(jnp.int32, x.shape, 1)
    vals = []
    idxs = []
    for _ in range(_K):
        m = jnp.max(x, axis=1, keepdims=True)
        sel = jnp.min(jnp.where(x == m, cols, n), axis=1, keepdims=True)
        vals.append(m)
        idxs.append(sel)
        x = jnp.where(cols == sel, -jnp.inf, x)
    vals_ref[...] = jnp.concatenate(vals, axis=1)
    idx_ref[...] = jnp.concatenate(idxs, axis=1)


def _topk(em2d):
    r, v = em2d.shape
    return pl.pallas_call(
        _topk_block,
        grid=(r // _ROWS,),
        in_specs=[pl.BlockSpec((_ROWS, v), lambda i: (i, 0))],
        out_specs=[
            pl.BlockSpec((_ROWS, _K), lambda i: (i, 0)),
            pl.BlockSpec((_ROWS, _K), lambda i: (i, 0)),
        ],
        out_shape=[
            jax.ShapeDtypeStruct((r, _K), jnp.float32),
            jax.ShapeDtypeStruct((r, _K), jnp.int32),
        ],
    )(em2d)


_CAP = 2048

_GDN = lax.GatherDimensionNumbers(
    offset_dims=(), collapsed_slice_dims=(0,), start_index_map=(0,))


def _gather16(v, idx):
    return lax.gather(v, idx[:, None], _GDN, (1,),
                      mode=lax.GatherScatterMode.PROMISE_IN_BOUNDS)


def _sc_compact(em2d):
    rows, v = em2d.shape
    info = plsc.get_sparse_core_info()
    nc, ns = info.num_cores, info.num_subcores
    nw = nc * ns
    rows_per_w = rows // nw
    nvec = v // 16
    mesh = plsc.VectorSubcoreMesh(core_axis_name="c", subcore_axis_name="s")

    @functools.partial(
        pl.kernel,
        mesh=mesh,
        out_type=[
            jax.ShapeDtypeStruct((rows, _CAP), jnp.float32),
            jax.ShapeDtypeStruct((rows, _CAP), jnp.int32),
        ],
        scratch_types=[
            pltpu.VMEM((v,), jnp.float32),
            pltpu.VMEM((_CAP + 16,), jnp.float32),
            pltpu.VMEM((_CAP + 16,), jnp.int32),
        ],
        compiler_params=pltpu.CompilerParams(needs_layout_passes=False),
    )
    def compact_kernel(em_hbm, val_hbm, pos_hbm, row_buf, val_buf, pos_buf):
        wid = lax.axis_index("s") * nc + lax.axis_index("c")
        lane = lax.broadcasted_iota(jnp.int32, (16,), 0)
        neg_inf = jnp.full((16,), -jnp.inf, jnp.float32)

        def bmax(vec):
            for s in (1, 2, 4, 8):
                vec = jnp.maximum(vec, _gather16(vec, jnp.bitwise_xor(lane, s)))
            return vec

        def row_body(r, carry):
            row = wid * rows_per_w + r
            pltpu.sync_copy(em_hbm.at[row], row_buf)

            def chunk_body(c, t0):
                acc = neg_inf
                for i in range(32):
                    acc = jnp.maximum(acc, row_buf[pl.ds(c * 512 + i * 16, 16)])
                return jnp.minimum(t0, bmax(acc))

            t0 = lax.fori_loop(0, 64, chunk_body,
                               jnp.full((16,), jnp.inf, jnp.float32))

            def init_body(i, carry2):
                val_buf[pl.ds(i * 16, 16)] = neg_inf
                return carry2

            lax.fori_loop(0, (_CAP + 16) // 16, init_body, 0)

            def comp_body(g, cnt_vec):
                for u in range(4):
                    i = g * 4 + u
                    x = row_buf[pl.ds(i * 16, 16)]
                    msk = x >= t0
                    posv = jnp.where(msk, lax.shift_left(cnt_vec, 4) + lane,
                                     _CAP + 15)
                    plsc.store_scatter(val_buf, [posv], x)
                    plsc.store_scatter(pos_buf, [posv], lane + i * 16)
                    cnt_vec = jnp.minimum(cnt_vec + msk.astype(jnp.int32),
                                          _CAP // 16 - 1)
                return cnt_vec

            lax.fori_loop(0, nvec // 4, comp_body, jnp.zeros((16,), jnp.int32))

            pltpu.sync_copy(val_buf.at[pl.ds(0, _CAP)], val_hbm.at[row])
            pltpu.sync_copy(pos_buf.at[pl.ds(0, _CAP)], pos_hbm.at[row])
            return carry

        lax.fori_loop(0, rows_per_w, row_body, 0)

    return compact_kernel(em2d)


def _sc_gather(table, cand_pos, sel_pos):
    info = plsc.get_sparse_core_info()
    nc, ns = info.num_cores, info.num_subcores
    nw = nc * ns
    rows, cap = cand_pos.shape
    n_idx = sel_pos.size
    d = table.shape[1]
    per_w = n_idx // nw
    rows_per_w = rows // nw
    k = n_idx // rows
    chunks = per_w // 128
    half = chunks // 2
    sel_flat = sel_pos.reshape(-1)
    mesh = plsc.VectorSubcoreMesh(core_axis_name="c", subcore_axis_name="s")

    @functools.partial(
        pl.kernel,
        mesh=mesh,
        out_type=jax.ShapeDtypeStruct((n_idx, d), jnp.float32),
        scratch_types=[
            pltpu.VMEM((rows_per_w, cap), jnp.int32),
            pltpu.VMEM((per_w,), jnp.int32),
            pltpu.VMEM((chunks, 128), jnp.int32),
            pltpu.VMEM((half * 128, d), jnp.float32),
            pltpu.SemaphoreType.DMA,
        ],
        compiler_params=pltpu.CompilerParams(needs_layout_passes=False),
    )
    def gather_kernel(table_hbm, cpos_hbm, sel_hbm, out_hbm,
                      cpos_v, sel_v, idx_v, rows_v, sem):
        wid = lax.axis_index("s") * nc + lax.axis_index("c")
        lane = lax.broadcasted_iota(jnp.int32, (16,), 0)
        pltpu.sync_copy(cpos_hbm.at[pl.ds(wid * rows_per_w, rows_per_w)], cpos_v)
        pltpu.sync_copy(sel_hbm.at[pl.ds(wid * per_w, per_w)], sel_v)
        kshift = k.bit_length() - 1
        for j in range(per_w // 16):
            slot = lane + j * 16
            row_loc = lax.shift_right_logical(slot, kshift)
            pos_v = sel_v[pl.ds(j * 16, 16)]
            real = plsc.load_gather(cpos_v, [row_loc, pos_v])
            idx_v[j // 8, pl.ds((j % 8) * 16, 16)] = real
        for h in range(2):
            copies = [
                pltpu.async_copy(
                    table_hbm.at[idx_v.at[h * half + j]],
                    rows_v.at[pl.ds(j * 128, 128)],
                    sem,
                )
                for j in range(half)
            ]
            for cp in copies:
                cp.wait()
            pltpu.sync_copy(
                rows_v,
                out_hbm.at[pl.ds(wid * per_w + h * half * 128, half * 128)],
            )

    return gather_kernel(table, cand_pos, sel_flat)


def _crf_block(emis_ref, st_ref, len_ref, out_ref):
    t_len, bk = emis_ref.shape
    b = bk // _K
    ii = lax.broadcasted_iota(jnp.int32, (_K, _K), 0)
    jj = lax.broadcasted_iota(jnp.int32, (_K, _K), 1)
    eye = (ii == jj).astype(jnp.float32)
    bb = lax.broadcasted_iota(jnp.int32, (b, bk), 0)
    mm = lax.broadcasted_iota(jnp.int32, (b, bk), 1)
    blk = (mm // _K == bb).astype(jnp.float32)
    lens = len_ref[...]

    def lse_rows(a):
        m = jnp.max(a, axis=1, keepdims=True)
        return m + jnp.log(jnp.sum(jnp.exp(a - m), axis=1, keepdims=True))

    def to_bk(row):
        return jnp.concatenate([row[:, i * _K:(i + 1) * _K] for i in range(b)],
                               axis=0)

    alpha = to_bk(emis_ref[0:1, :])
    res = lse_rows(alpha)
    for t in range(1, t_len):
        alpha_t = lax.dot_general(eye, alpha, (((1,), (1,)), ((), ())),
                                  precision=lax.Precision.HIGHEST,
                                  preferred_element_type=jnp.float32)
        alpha_bcast = lax.dot_general(alpha_t, blk, (((1,), (0,)), ((), ())),
                                      precision=lax.Precision.HIGHEST,
                                      preferred_element_type=jnp.float32)
        trans = jnp.concatenate(
            [lax.dot_general(st_ref[i, t - 1], st_ref[i, t],
                             (((1,), (1,)), ((), ())),
                             preferred_element_type=jnp.float32)
             for i in range(b)], axis=1)
        mat = trans + alpha_bcast
        m = jnp.max(mat, axis=0, keepdims=True)
        new_row = (m + jnp.log(jnp.sum(jnp.exp(mat - m), axis=0, keepdims=True))
                   + emis_ref[t:t + 1, :])
        alpha = to_bk(new_row)
        res = jnp.where(lens - 1 == t, lse_rows(alpha), res)
    out_ref[...] = res


def _crf(emis3, st4, lens3):
    b, t, _ = emis3.shape
    emis_tb = jnp.transpose(emis3, (1, 0, 2)).reshape(t, b * _K)
    lens2 = lens3.reshape(b, 1)
    return pl.pallas_call(
        _crf_block,
        out_shape=jax.ShapeDtypeStruct((b, 1), jnp.float32),
    )(emis_tb, st4, lens2)


def kernel(state_matrix, emission_potentials, seq_lens, sum_size):
    b, t, v = emission_potentials.shape
    em2d = emission_potentials.reshape(b * t, v)
    cand_val, cand_pos = _sc_compact(em2d)
    vals, sel_pos = _topk(cand_val)
    d = state_matrix.shape[1]
    table_pad = jnp.pad(state_matrix, ((0, 0), (0, 128 - d)))
    sampled = _sc_gather(table_pad, cand_pos, sel_pos)[:, :d]
    emis3 = vals.reshape(b, t, _K)
    st4 = sampled.reshape(b, t, _K, d)
    lens3 = seq_lens.reshape(b, 1, 1)
    out3 = _crf(emis3, st4, lens3)
    return out3.reshape(b)
